# Initial kernel scaffold; baseline (speedup 1.0000x reference)
import jax
import jax.numpy as jnp
from jax.experimental import pallas as pl


def kernel(x, edge, adj, batch, lin_adj_W, lin_adj_b, W1, att_src1, att_dst1, bias1, W2, att_src2, att_dst2, bias2, ln_g, ln_b, lin1_W, lin1_b):
    raise NotImplementedError("write your pallas kernel here")


# speedup vs baseline: 3.7086x; 3.7086x over previous
import functools

import jax
import jax.numpy as jnp
from jax.experimental import pallas as pl

N = 400
HID = 512
C_OUT = 128
NUM_GRAPHS = 8

_bf16 = jnp.bfloat16
_f32 = jnp.float32


def _dot(a, b):
    return jnp.dot(a.astype(_bf16), b.astype(_bf16), preferred_element_type=_f32)


def _fused_kernel(adjT_ref, ladjW_ref, ladjb_ref, x_ref, w1t_ref, as1_ref,
                  ad1_ref, b1_ref, w2t_ref, as2_ref, ad2_ref, b2_ref,
                  lng_ref, lnb_ref, lin1t_ref, lin1b_ref, batch_ref, out_ref):
    aT = jnp.dot(ladjW_ref[...], adjT_ref[...], preferred_element_type=_f32)
    aT = aT + ladjb_ref[...]
    keep = jax.nn.sigmoid(aT) > 0.6
    rows = jax.lax.broadcasted_iota(jnp.int32, (N, N), 0)
    cols = jax.lax.broadcasted_iota(jnp.int32, (N, N), 1)
    mask = keep | (rows == cols)
    neg = jnp.float32(-1e9)

    def gat(h_in, wt_ref, a_s_ref, a_d_ref, b_ref):
        h = jnp.dot(h_in.astype(_bf16), wt_ref[...],
                    preferred_element_type=_f32)
        hb = h.astype(_bf16).astype(_f32)
        a_s = jax.lax.dot_general(a_s_ref[...].astype(_f32), hb,
                                  (((1,), (1,)), ((), ())),
                                  preferred_element_type=_f32)
        a_d = jax.lax.dot_general(hb, a_d_ref[...].astype(_f32),
                                  (((1,), (1,)), ((), ())),
                                  preferred_element_type=_f32)
        e = a_d + a_s
        e = jnp.where(e >= 0, e, 0.2 * e)
        e = jnp.where(mask, e, neg)
        m = jnp.max(e, axis=1, keepdims=True)
        p = jnp.exp(e - m)
        s = jnp.sum(p, axis=1, keepdims=True)
        attn = p / s
        return _dot(attn, h) + b_ref[...]

    def ln_relu(h):
        mu = jnp.mean(h, axis=1, keepdims=True)
        d = h - mu
        var = jnp.mean(d * d, axis=1, keepdims=True)
        h = d * jax.lax.rsqrt(var + 1e-5) * lng_ref[...] + lnb_ref[...]
        return jnp.maximum(h, 0.0)

    h = gat(x_ref[...], w1t_ref, as1_ref, ad1_ref, b1_ref)
    h = ln_relu(h)
    h = gat(h, w2t_ref, as2_ref, ad2_ref, b2_ref)
    h = ln_relu(h)
    h = gat(h, w2t_ref, as2_ref, ad2_ref, b2_ref)

    gi = jax.lax.broadcasted_iota(jnp.int32, (NUM_GRAPHS, N), 0)
    seg = (batch_ref[...] == gi).astype(_f32)
    cnt = jnp.sum(seg, axis=1, keepdims=True)
    sums = jnp.dot(seg, h, preferred_element_type=_f32,
                   precision=jax.lax.Precision.HIGHEST)
    pooled = sums / jnp.maximum(cnt, 1.0)
    out_ref[...] = _dot(pooled, lin1t_ref[...]) + lin1b_ref[...]


@functools.partial(jax.jit, static_argnames=())
def kernel(x, edge, adj, batch, lin_adj_W, lin_adj_b, W1, att_src1, att_dst1,
           bias1, W2, att_src2, att_dst2, bias2, ln_g, ln_b, lin1_W, lin1_b):
    del edge
    args = (
        adj.T.astype(_bf16),
        lin_adj_W.astype(_bf16),
        lin_adj_b.reshape(N, 1),
        x.astype(_bf16),
        W1.T.astype(_bf16),
        att_src1.reshape(1, HID).astype(_bf16),
        att_dst1.reshape(1, HID).astype(_bf16),
        bias1.reshape(1, HID),
        W2.T.astype(_bf16),
        att_src2.reshape(1, HID).astype(_bf16),
        att_dst2.reshape(1, HID).astype(_bf16),
        bias2.reshape(1, HID),
        ln_g.reshape(1, HID),
        ln_b.reshape(1, HID),
        lin1_W.T.astype(_bf16),
        lin1_b.reshape(1, C_OUT),
        batch.reshape(1, N).astype(jnp.int32),
    )
    return pl.pallas_call(
        _fused_kernel,
        out_shape=jax.ShapeDtypeStruct((NUM_GRAPHS, C_OUT), jnp.float32),
    )(*args)


# speedup vs baseline: 7.5359x; 2.0320x over previous
import functools

import jax
import jax.numpy as jnp
from jax.experimental import pallas as pl

N = 400
HID = 512
C_OUT = 128
NUM_GRAPHS = 8

_bf16 = jnp.bfloat16
_f32 = jnp.float32

_DN_T = (((1,), (1,)), ((), ()))


def _dot_t(a, b):
    return jax.lax.dot_general(a.astype(_bf16), b.astype(_bf16), _DN_T,
                               preferred_element_type=_f32)


def _fused_kernel(adj_ref, ladjW_ref, ladjb_ref, x_ref, w1_ref, as1_ref,
                  ad1_ref, b1_ref, w2_ref, as2_ref, ad2_ref, b2_ref,
                  lng_ref, lnb_ref, lin1_ref, lin1b_ref, batch_ref, out_ref):
    aT = _dot_t(ladjW_ref[...], adj_ref[...])
    aT = aT + ladjb_ref[...]
    keep = jax.nn.sigmoid(aT) > 0.6
    rows = jax.lax.broadcasted_iota(jnp.int32, (N, N), 0)
    cols = jax.lax.broadcasted_iota(jnp.int32, (N, N), 1)
    mask = keep | (rows == cols)
    neg = jnp.float32(-1e9)

    def gat(h_in, w_ref, a_s_ref, a_d_ref, b_ref):
        h = _dot_t(h_in, w_ref[...])
        hb = h.astype(_bf16).astype(_f32)
        a_s = jax.lax.dot_general(
            a_s_ref[...].astype(_bf16).astype(_f32), hb, _DN_T,
            preferred_element_type=_f32)
        a_d = jax.lax.dot_general(
            hb, a_d_ref[...].astype(_bf16).astype(_f32), _DN_T,
            preferred_element_type=_f32)
        e = a_d + a_s
        e = jnp.where(e >= 0, e, 0.2 * e)
        e = jnp.where(mask, e, neg)
        m = jnp.max(e, axis=1, keepdims=True)
        p = jnp.exp(e - m)
        s = jnp.sum(p, axis=1, keepdims=True)
        attn = p / s
        return jnp.dot(attn.astype(_bf16), h.astype(_bf16),
                       preferred_element_type=_f32) + b_ref[...]

    def ln_relu(h):
        mu = jnp.mean(h, axis=1, keepdims=True)
        d = h - mu
        var = jnp.mean(d * d, axis=1, keepdims=True)
        h = d * jax.lax.rsqrt(var + 1e-5) * lng_ref[...] + lnb_ref[...]
        return jnp.maximum(h, 0.0)

    h = gat(x_ref[...], w1_ref, as1_ref, ad1_ref, b1_ref)
    h = ln_relu(h)
    h = gat(h, w2_ref, as2_ref, ad2_ref, b2_ref)
    h = ln_relu(h)
    h = gat(h, w2_ref, as2_ref, ad2_ref, b2_ref)

    gi = jax.lax.broadcasted_iota(jnp.int32, (NUM_GRAPHS, N), 0)
    seg = (batch_ref[...] == gi).astype(_f32)
    cnt = jnp.sum(seg, axis=1, keepdims=True)
    sums = jnp.dot(seg, h, preferred_element_type=_f32,
                   precision=jax.lax.Precision.HIGHEST)
    pooled = sums / jnp.maximum(cnt, 1.0)
    out_ref[...] = _dot_t(pooled, lin1_ref[...]) + lin1b_ref[...]


@functools.partial(jax.jit, static_argnames=())
def kernel(x, edge, adj, batch, lin_adj_W, lin_adj_b, W1, att_src1, att_dst1,
           bias1, W2, att_src2, att_dst2, bias2, ln_g, ln_b, lin1_W, lin1_b):
    del edge
    args = (
        adj,
        lin_adj_W,
        lin_adj_b.reshape(N, 1),
        x,
        W1,
        att_src1.reshape(1, HID),
        att_dst1.reshape(1, HID),
        bias1.reshape(1, HID),
        W2,
        att_src2.reshape(1, HID),
        att_dst2.reshape(1, HID),
        bias2.reshape(1, HID),
        ln_g.reshape(1, HID),
        ln_b.reshape(1, HID),
        lin1_W,
        lin1_b.reshape(1, C_OUT),
        batch.reshape(1, N).astype(jnp.int32),
    )
    return pl.pallas_call(
        _fused_kernel,
        out_shape=jax.ShapeDtypeStruct((NUM_GRAPHS, C_OUT), jnp.float32),
    )(*args)
